# Initial kernel scaffold; baseline (speedup 1.0000x reference)
import jax
import jax.numpy as jnp
from jax.experimental import pallas as pl


def kernel(x, edge_index, Wz, bz, Wr, br, Wh, bh, lzW, lzb, lrW, lrb, lhW, lhb, linW, linb):
    raise NotImplementedError("write your pallas kernel here")


# speedup vs baseline: 100.4788x; 100.4788x over previous
import functools

import jax
import jax.numpy as jnp
from jax import lax
from jax.experimental import pallas as pl
from jax.experimental.pallas import tpu as pltpu
from jax.experimental.pallas import tpu_sc as plsc

N_NODES = 100000
IN_C = 8
HID = 32

NC = 2
NS = 16
NW = NC * NS

CHUNK = 128
N_PAD = 100096
STRIPE = N_PAD // NS
NB = N_PAD // 16

E = 3200000
E_PAD = 3211264
DROW = E_PAD // CHUNK
ROWS_PT = DROW // NW
SB2 = 4
NBLK2 = ROWS_PT // SB2
SB1 = 8
NBLK1 = ROWS_PT // SB1

_mesh = plsc.VectorSubcoreMesh(core_axis_name="c", subcore_axis_name="s")
_no_tc_tiling = pltpu.CompilerParams(use_tc_tiling_on_sc=False)


@functools.partial(
    pl.kernel,
    out_type=jax.ShapeDtypeStruct((NC * N_PAD,), jnp.float32),
    mesh=_mesh,
    scratch_types=[
        pltpu.VMEM((SB1, CHUNK), jnp.int32),
        pltpu.VMEM((SB1, CHUNK), jnp.int32),
        pltpu.VMEM((CHUNK,), jnp.float32),
        pltpu.VMEM((STRIPE,), jnp.float32),
        pltpu.SemaphoreType.DMA,
        pltpu.SemaphoreType.DMA,
        pltpu.SemaphoreType.DMA,
        pltpu.VMEM_SHARED((N_PAD,), jnp.float32),
    ],
    compiler_params=_no_tc_tiling,
)
def _count_kernel(ei2_hbm, z1_hbm, ones_hbm, cnt_hbm, diA, diB, ones,
                  stage, semA, semB, semS, cnt_sh):
    c = lax.axis_index("c")
    s = lax.axis_index("s")
    w = s * NC + c
    row0 = DROW + w * ROWS_PT
    pltpu.sync_copy(z1_hbm, stage)
    pltpu.sync_copy(stage, cnt_sh.at[pl.ds(s * STRIPE, STRIPE)])
    pltpu.sync_copy(ones_hbm, ones)
    plsc.subcore_barrier()

    def fetch(blk, di, sem):
        pltpu.async_copy(ei2_hbm.at[pl.ds(row0 + blk * SB1, SB1), :], di, sem)

    def process(blk, di, sem):
        pltpu.make_async_copy(ei2_hbm.at[pl.ds(0, SB1), :], di, sem).wait()
        scat = [pltpu.async_copy(ones, cnt_sh.at[di.at[k]], semS, add=True)
                for k in range(SB1)]
        for d in scat:
            d.wait()

        @pl.when(blk + 2 < NBLK1)
        def _():
            fetch(blk + 2, di, sem)

    fetch(0, diA, semA)
    fetch(1, diB, semB)

    def body(i, carry):
        process(2 * i, diA, semA)
        process(2 * i + 1, diB, semB)
        return carry

    lax.fori_loop(0, NBLK1 // 2, body, 0)
    plsc.subcore_barrier()
    pltpu.sync_copy(cnt_sh.at[pl.ds(s * STRIPE, STRIPE)], stage)
    pltpu.sync_copy(stage, cnt_hbm.at[pl.ds(c * N_PAD + s * STRIPE, STRIPE)])


@functools.partial(
    pl.kernel,
    out_type=jax.ShapeDtypeStruct((NC, N_PAD, IN_C), jnp.float32),
    mesh=_mesh,
    scratch_types=[
        pltpu.VMEM((SB2, CHUNK), jnp.int32),
        pltpu.VMEM((SB2, CHUNK), jnp.int32),
        pltpu.VMEM((SB2, CHUNK, IN_C), jnp.float32),
        pltpu.VMEM((SB2, CHUNK), jnp.int32),
        pltpu.VMEM((SB2, CHUNK), jnp.int32),
        pltpu.VMEM((SB2, CHUNK, IN_C), jnp.float32),
        pltpu.VMEM((STRIPE, IN_C), jnp.float32),
        pltpu.SemaphoreType.DMA,
        pltpu.SemaphoreType.DMA,
        pltpu.SemaphoreType.DMA,
        pltpu.SemaphoreType.DMA,
        pltpu.VMEM_SHARED((N_PAD, IN_C), jnp.float32),
    ],
    compiler_params=_no_tc_tiling,
)
def _agg_kernel(ei2_hbm, xs_hbm, z2_hbm, out_hbm, siA, diA, rA, siB, diB,
                rB, stage, semA, semB, semG, semS, acc):
    c = lax.axis_index("c")
    s = lax.axis_index("s")
    w = s * NC + c
    row0 = w * ROWS_PT
    pltpu.sync_copy(z2_hbm, stage)
    pltpu.sync_copy(stage, acc.at[pl.ds(s * STRIPE, STRIPE), :])
    plsc.subcore_barrier()

    def fetch(blk, si, di, sem):
        pltpu.async_copy(ei2_hbm.at[pl.ds(row0 + blk * SB2, SB2), :], si, sem)
        pltpu.async_copy(ei2_hbm.at[pl.ds(DROW + row0 + blk * SB2, SB2), :],
                         di, sem)

    def process(blk, si, di, rr, sem):
        pltpu.make_async_copy(ei2_hbm.at[pl.ds(0, SB2), :], si, sem).wait()
        pltpu.make_async_copy(ei2_hbm.at[pl.ds(0, SB2), :], di, sem).wait()
        gat = [pltpu.async_copy(xs_hbm.at[si.at[k]], rr.at[k], semG)
               for k in range(SB2)]
        scat = []
        for k in range(SB2):
            gat[k].wait()
            scat.append(pltpu.async_copy(rr.at[k], acc.at[di.at[k]], semS,
                                         add=True))
        for d in scat:
            d.wait()

        @pl.when(blk + 2 < NBLK2)
        def _():
            fetch(blk + 2, si, di, sem)

    fetch(0, siA, diA, semA)
    fetch(1, siB, diB, semB)

    def body(i, carry):
        process(2 * i, siA, diA, rA, semA)
        process(2 * i + 1, siB, diB, rB, semB)
        return carry

    lax.fori_loop(0, NBLK2 // 2, body, 0)
    plsc.subcore_barrier()
    pltpu.sync_copy(acc.at[pl.ds(s * STRIPE, STRIPE), :], stage)
    pltpu.sync_copy(stage, out_hbm.at[c, pl.ds(s * STRIPE, STRIPE), :])


def _prep_body(cnt2_ref, x_ref, dinv_ref, xs_ref):
    cnt = cnt2_ref[0] + cnt2_ref[1]
    dinv = lax.rsqrt(cnt + 1.0)
    dinv_ref[...] = dinv
    xs_ref[...] = x_ref[...] * dinv


def _tail_body(t2_ref, xs_ref, dinv_ref, mz_ref, cz_ref, mh_ref, ch_ref,
               w_ref, b_ref, out_ref):
    s = dinv_ref[...] * (t2_ref[0] + t2_ref[1] + xs_ref[...])
    gz = jnp.dot(s, mz_ref[...], preferred_element_type=jnp.float32,
                 precision=lax.Precision.HIGHEST) + cz_ref[...]
    gh = jnp.dot(s, mh_ref[...], preferred_element_type=jnp.float32,
                 precision=lax.Precision.HIGHEST) + ch_ref[...]
    z = jax.nn.sigmoid(gz)
    ht = jnp.tanh(gh)
    y = (1.0 - z) * ht
    out_ref[...] = jnp.dot(y, w_ref[...], preferred_element_type=jnp.float32,
                           precision=lax.Precision.HIGHEST) + b_ref[...]


def kernel(x, edge_index, Wz, bz, Wr, br, Wh, bh, lzW, lzb, lrW, lrb, lhW,
           lhb, linW, linb):
    n = x.shape[0]
    assert n == N_NODES and edge_index.shape[1] == E

    Az = lzW[:, :HID].T
    Ah = lhW[:, :HID].T
    Mz = Wz @ Az
    cz = (bz @ Az + lzb).reshape(1, HID)
    Mh = Wh @ Ah
    ch = (bh @ Ah + lhb).reshape(1, HID)
    wv = linW.reshape(HID, 1)
    bs = linb.reshape(1, 1)

    x_pad = jnp.zeros((N_PAD, IN_C), jnp.float32).at[:n].set(x)
    z1 = jnp.zeros((STRIPE,), jnp.float32)
    z2 = jnp.zeros((STRIPE, IN_C), jnp.float32)
    ones = jnp.ones((CHUNK,), jnp.float32)

    ei2 = jnp.pad(edge_index, ((0, 0), (0, E_PAD - E)),
                  constant_values=n).reshape(2 * DROW, CHUNK)

    cnt2 = _count_kernel(ei2, z1, ones)

    grid = (N_PAD // NB,)
    dinv, xs = pl.pallas_call(
        _prep_body,
        grid=grid,
        in_specs=[
            pl.BlockSpec((NC, NB, 1), lambda i: (0, i, 0)),
            pl.BlockSpec((NB, IN_C), lambda i: (i, 0)),
        ],
        out_specs=[
            pl.BlockSpec((NB, 1), lambda i: (i, 0)),
            pl.BlockSpec((NB, IN_C), lambda i: (i, 0)),
        ],
        out_shape=[
            jax.ShapeDtypeStruct((N_PAD, 1), jnp.float32),
            jax.ShapeDtypeStruct((N_PAD, IN_C), jnp.float32),
        ],
    )(cnt2.reshape(NC, N_PAD, 1), x_pad)

    t2 = _agg_kernel(ei2, xs, z2)

    out = pl.pallas_call(
        _tail_body,
        grid=grid,
        in_specs=[
            pl.BlockSpec((NC, NB, IN_C), lambda i: (0, i, 0)),
            pl.BlockSpec((NB, IN_C), lambda i: (i, 0)),
            pl.BlockSpec((NB, 1), lambda i: (i, 0)),
            pl.BlockSpec((IN_C, HID), lambda i: (0, 0)),
            pl.BlockSpec((1, HID), lambda i: (0, 0)),
            pl.BlockSpec((IN_C, HID), lambda i: (0, 0)),
            pl.BlockSpec((1, HID), lambda i: (0, 0)),
            pl.BlockSpec((HID, 1), lambda i: (0, 0)),
            pl.BlockSpec((1, 1), lambda i: (0, 0)),
        ],
        out_specs=pl.BlockSpec((NB, 1), lambda i: (i, 0)),
        out_shape=jax.ShapeDtypeStruct((N_PAD, 1), jnp.float32),
    )(t2, xs, dinv, Mz, cz, Mh, ch, wv, bs)

    return out[:n, 0]


# speedup vs baseline: 106.4821x; 1.0597x over previous
import functools

import jax
import jax.numpy as jnp
from jax import lax
from jax.experimental import pallas as pl
from jax.experimental.pallas import tpu as pltpu
from jax.experimental.pallas import tpu_sc as plsc

N_NODES = 100000
IN_C = 8
HID = 32

NC = 2
NS = 16
NW = NC * NS

CHUNK = 128
N_PAD = 100096
STRIPE = N_PAD // NS
NB = N_PAD // 16

E = 3200000
DROW = E // CHUNK
ROWS_PT = 784
SB2 = 4
SB1 = 8

_mesh = plsc.VectorSubcoreMesh(core_axis_name="c", subcore_axis_name="s")
_no_tc_tiling = pltpu.CompilerParams(use_tc_tiling_on_sc=False)


@functools.partial(
    pl.kernel,
    out_type=jax.ShapeDtypeStruct((NC * N_PAD,), jnp.float32),
    mesh=_mesh,
    scratch_types=[
        pltpu.VMEM((SB1, CHUNK), jnp.int32),
        pltpu.VMEM((SB1, CHUNK), jnp.int32),
        pltpu.VMEM((CHUNK,), jnp.float32),
        pltpu.VMEM((STRIPE,), jnp.float32),
        pltpu.SemaphoreType.DMA,
        pltpu.SemaphoreType.DMA,
        pltpu.SemaphoreType.DMA,
        pltpu.VMEM_SHARED((N_PAD,), jnp.float32),
    ],
    compiler_params=_no_tc_tiling,
)
def _count_kernel(ei2_hbm, z1_hbm, ones_hbm, cnt_hbm, diA, diB, ones,
                  stage, semA, semB, semS, cnt_sh):
    c = lax.axis_index("c")
    s = lax.axis_index("s")
    w = s * NC + c
    row0 = DROW + w * ROWS_PT
    nblk = jnp.minimum(ROWS_PT, DROW - w * ROWS_PT) // SB1
    pltpu.sync_copy(z1_hbm, stage)
    pltpu.sync_copy(stage, cnt_sh.at[pl.ds(s * STRIPE, STRIPE)])
    pltpu.sync_copy(ones_hbm, ones)
    plsc.subcore_barrier()

    def fetch(blk, di, sem):
        pltpu.async_copy(ei2_hbm.at[pl.ds(row0 + blk * SB1, SB1), :], di, sem)

    def process(blk, di, sem):
        pltpu.make_async_copy(ei2_hbm.at[pl.ds(0, SB1), :], di, sem).wait()
        scat = [pltpu.async_copy(ones, cnt_sh.at[di.at[k]], semS, add=True)
                for k in range(SB1)]
        for d in scat:
            d.wait()

        @pl.when(blk + 2 < nblk)
        def _():
            fetch(blk + 2, di, sem)

    fetch(0, diA, semA)
    fetch(1, diB, semB)

    def body(i, carry):
        process(2 * i, diA, semA)
        process(2 * i + 1, diB, semB)
        return carry

    lax.fori_loop(0, nblk // 2, body, 0)

    @pl.when(nblk % 2 == 1)
    def _():
        process(nblk - 1, diA, semA)

    plsc.subcore_barrier()
    pltpu.sync_copy(cnt_sh.at[pl.ds(s * STRIPE, STRIPE)], stage)
    pltpu.sync_copy(stage, cnt_hbm.at[pl.ds(c * N_PAD + s * STRIPE, STRIPE)])


@functools.partial(
    pl.kernel,
    out_type=jax.ShapeDtypeStruct((NC, N_PAD, IN_C), jnp.float32),
    mesh=_mesh,
    scratch_types=[
        pltpu.VMEM((SB2, CHUNK), jnp.int32),
        pltpu.VMEM((SB2, CHUNK), jnp.int32),
        pltpu.VMEM((SB2, CHUNK, IN_C), jnp.float32),
        pltpu.VMEM((SB2, CHUNK), jnp.int32),
        pltpu.VMEM((SB2, CHUNK), jnp.int32),
        pltpu.VMEM((SB2, CHUNK, IN_C), jnp.float32),
        pltpu.VMEM((STRIPE, IN_C), jnp.float32),
        pltpu.SemaphoreType.DMA,
        pltpu.SemaphoreType.DMA,
        pltpu.SemaphoreType.DMA,
        pltpu.SemaphoreType.DMA,
        pltpu.VMEM_SHARED((N_PAD, IN_C), jnp.float32),
    ],
    compiler_params=_no_tc_tiling,
)
def _agg_kernel(ei2_hbm, xs_hbm, z2_hbm, out_hbm, siA, diA, rA, siB, diB,
                rB, stage, semA, semB, semG, semS, acc):
    c = lax.axis_index("c")
    s = lax.axis_index("s")
    w = s * NC + c
    row0 = w * ROWS_PT
    nblk = jnp.minimum(ROWS_PT, DROW - w * ROWS_PT) // SB2
    pltpu.sync_copy(z2_hbm, stage)
    pltpu.sync_copy(stage, acc.at[pl.ds(s * STRIPE, STRIPE), :])
    plsc.subcore_barrier()

    def fetch(blk, si, di, sem):
        pltpu.async_copy(ei2_hbm.at[pl.ds(row0 + blk * SB2, SB2), :], si, sem)
        pltpu.async_copy(ei2_hbm.at[pl.ds(DROW + row0 + blk * SB2, SB2), :],
                         di, sem)

    def process(blk, si, di, rr, sem):
        pltpu.make_async_copy(ei2_hbm.at[pl.ds(0, SB2), :], si, sem).wait()
        pltpu.make_async_copy(ei2_hbm.at[pl.ds(0, SB2), :], di, sem).wait()
        gat = [pltpu.async_copy(xs_hbm.at[si.at[k]], rr.at[k], semG)
               for k in range(SB2)]
        scat = []
        for k in range(SB2):
            gat[k].wait()
            scat.append(pltpu.async_copy(rr.at[k], acc.at[di.at[k]], semS,
                                         add=True))
        for d in scat:
            d.wait()

        @pl.when(blk + 2 < nblk)
        def _():
            fetch(blk + 2, si, di, sem)

    fetch(0, siA, diA, semA)
    fetch(1, siB, diB, semB)

    def body(i, carry):
        process(2 * i, siA, diA, rA, semA)
        process(2 * i + 1, siB, diB, rB, semB)
        return carry

    lax.fori_loop(0, nblk // 2, body, 0)

    @pl.when(nblk % 2 == 1)
    def _():
        process(nblk - 1, siA, diA, rA, semA)

    plsc.subcore_barrier()
    pltpu.sync_copy(acc.at[pl.ds(s * STRIPE, STRIPE), :], stage)
    pltpu.sync_copy(stage, out_hbm.at[c, pl.ds(s * STRIPE, STRIPE), :])


def _prep_body(cnt2_ref, x_ref, dinv_ref, xs_ref):
    cnt = cnt2_ref[0] + cnt2_ref[1]
    dinv = lax.rsqrt(cnt + 1.0)
    dinv_ref[...] = dinv
    xs_ref[...] = x_ref[...] * dinv


def _tail_body(t2_ref, xs_ref, dinv_ref, mz_ref, cz_ref, mh_ref, ch_ref,
               w_ref, b_ref, out_ref):
    s = dinv_ref[...] * (t2_ref[0] + t2_ref[1] + xs_ref[...])
    gz = jnp.dot(s, mz_ref[...], preferred_element_type=jnp.float32,
                 precision=lax.Precision.HIGHEST) + cz_ref[...]
    gh = jnp.dot(s, mh_ref[...], preferred_element_type=jnp.float32,
                 precision=lax.Precision.HIGHEST) + ch_ref[...]
    z = jax.nn.sigmoid(gz)
    ht = jnp.tanh(gh)
    y = (1.0 - z) * ht
    out_ref[...] = jnp.dot(y, w_ref[...], preferred_element_type=jnp.float32,
                           precision=lax.Precision.HIGHEST) + b_ref[...]


def kernel(x, edge_index, Wz, bz, Wr, br, Wh, bh, lzW, lzb, lrW, lrb, lhW,
           lhb, linW, linb):
    n = x.shape[0]
    assert n == N_NODES and edge_index.shape[1] == E

    Az = lzW[:, :HID].T
    Ah = lhW[:, :HID].T
    Mz = Wz @ Az
    cz = (bz @ Az + lzb).reshape(1, HID)
    Mh = Wh @ Ah
    ch = (bh @ Ah + lhb).reshape(1, HID)
    wv = linW.reshape(HID, 1)
    bs = linb.reshape(1, 1)

    x_pad = jnp.zeros((N_PAD, IN_C), jnp.float32).at[:n].set(x)
    z1 = jnp.zeros((STRIPE,), jnp.float32)
    z2 = jnp.zeros((STRIPE, IN_C), jnp.float32)
    ones = jnp.ones((CHUNK,), jnp.float32)

    ei2 = edge_index.reshape(2 * DROW, CHUNK)

    cnt2 = _count_kernel(ei2, z1, ones)

    grid = (N_PAD // NB,)
    dinv, xs = pl.pallas_call(
        _prep_body,
        grid=grid,
        in_specs=[
            pl.BlockSpec((NC, NB, 1), lambda i: (0, i, 0)),
            pl.BlockSpec((NB, IN_C), lambda i: (i, 0)),
        ],
        out_specs=[
            pl.BlockSpec((NB, 1), lambda i: (i, 0)),
            pl.BlockSpec((NB, IN_C), lambda i: (i, 0)),
        ],
        out_shape=[
            jax.ShapeDtypeStruct((N_PAD, 1), jnp.float32),
            jax.ShapeDtypeStruct((N_PAD, IN_C), jnp.float32),
        ],
    )(cnt2.reshape(NC, N_PAD, 1), x_pad)

    t2 = _agg_kernel(ei2, xs, z2)

    out = pl.pallas_call(
        _tail_body,
        grid=grid,
        in_specs=[
            pl.BlockSpec((NC, NB, IN_C), lambda i: (0, i, 0)),
            pl.BlockSpec((NB, IN_C), lambda i: (i, 0)),
            pl.BlockSpec((NB, 1), lambda i: (i, 0)),
            pl.BlockSpec((IN_C, HID), lambda i: (0, 0)),
            pl.BlockSpec((1, HID), lambda i: (0, 0)),
            pl.BlockSpec((IN_C, HID), lambda i: (0, 0)),
            pl.BlockSpec((1, HID), lambda i: (0, 0)),
            pl.BlockSpec((HID, 1), lambda i: (0, 0)),
            pl.BlockSpec((1, 1), lambda i: (0, 0)),
        ],
        out_specs=pl.BlockSpec((NB, 1), lambda i: (i, 0)),
        out_shape=jax.ShapeDtypeStruct((N_PAD, 1), jnp.float32),
    )(t2, xs, dinv, Mz, cz, Mh, ch, wv, bs)

    return out[:n, 0]


# speedup vs baseline: 114.5098x; 1.0754x over previous
import functools

import jax
import jax.numpy as jnp
from jax import lax
from jax.experimental import pallas as pl
from jax.experimental.pallas import tpu as pltpu
from jax.experimental.pallas import tpu_sc as plsc

N_NODES = 100000
IN_C = 8
HID = 32

NC = 2
NS = 16
NW = NC * NS

CHUNK = 256
N_PAD = 100096
STRIPE = N_PAD // NS
NB = N_PAD // 16

E = 3200000
DROW = E // CHUNK
ROWS_PT = 392
SB2 = 4
SB1 = 4

_mesh = plsc.VectorSubcoreMesh(core_axis_name="c", subcore_axis_name="s")
_no_tc_tiling = pltpu.CompilerParams(use_tc_tiling_on_sc=False)


@functools.partial(
    pl.kernel,
    out_type=jax.ShapeDtypeStruct((NC * N_PAD,), jnp.float32),
    mesh=_mesh,
    scratch_types=[
        pltpu.VMEM((SB1, CHUNK), jnp.int32),
        pltpu.VMEM((SB1, CHUNK), jnp.int32),
        pltpu.VMEM((CHUNK,), jnp.float32),
        pltpu.VMEM((STRIPE,), jnp.float32),
        pltpu.SemaphoreType.DMA,
        pltpu.SemaphoreType.DMA,
        pltpu.SemaphoreType.DMA,
        pltpu.VMEM_SHARED((N_PAD,), jnp.float32),
    ],
    compiler_params=_no_tc_tiling,
)
def _count_kernel(ei2_hbm, z1_hbm, ones_hbm, cnt_hbm, diA, diB, ones,
                  stage, semA, semB, semS, cnt_sh):
    c = lax.axis_index("c")
    s = lax.axis_index("s")
    w = s * NC + c
    row0 = DROW + w * ROWS_PT
    nblk = jnp.minimum(ROWS_PT, DROW - w * ROWS_PT) // SB1
    pltpu.sync_copy(z1_hbm, stage)
    pltpu.sync_copy(stage, cnt_sh.at[pl.ds(s * STRIPE, STRIPE)])
    pltpu.sync_copy(ones_hbm, ones)
    plsc.subcore_barrier()

    def fetch(blk, di, sem):
        pltpu.async_copy(ei2_hbm.at[pl.ds(row0 + blk * SB1, SB1), :], di, sem)

    def process(blk, di, sem):
        pltpu.make_async_copy(ei2_hbm.at[pl.ds(0, SB1), :], di, sem).wait()
        scat = [pltpu.async_copy(ones, cnt_sh.at[di.at[k]], semS, add=True)
                for k in range(SB1)]
        for d in scat:
            d.wait()

        @pl.when(blk + 2 < nblk)
        def _():
            fetch(blk + 2, di, sem)

    fetch(0, diA, semA)
    fetch(1, diB, semB)

    def body(i, carry):
        process(2 * i, diA, semA)
        process(2 * i + 1, diB, semB)
        return carry

    lax.fori_loop(0, nblk // 2, body, 0)

    @pl.when(nblk % 2 == 1)
    def _():
        process(nblk - 1, diA, semA)

    plsc.subcore_barrier()
    pltpu.sync_copy(cnt_sh.at[pl.ds(s * STRIPE, STRIPE)], stage)
    pltpu.sync_copy(stage, cnt_hbm.at[pl.ds(c * N_PAD + s * STRIPE, STRIPE)])


@functools.partial(
    pl.kernel,
    out_type=jax.ShapeDtypeStruct((NC, N_PAD, IN_C), jnp.float32),
    mesh=_mesh,
    scratch_types=[
        pltpu.VMEM((SB2, CHUNK), jnp.int32),
        pltpu.VMEM((SB2, CHUNK), jnp.int32),
        pltpu.VMEM((SB2, CHUNK, IN_C), jnp.float32),
        pltpu.VMEM((SB2, CHUNK), jnp.int32),
        pltpu.VMEM((SB2, CHUNK), jnp.int32),
        pltpu.VMEM((SB2, CHUNK, IN_C), jnp.float32),
        pltpu.VMEM((STRIPE, IN_C), jnp.float32),
        pltpu.SemaphoreType.DMA,
        pltpu.SemaphoreType.DMA,
        pltpu.SemaphoreType.DMA,
        pltpu.SemaphoreType.DMA,
        pltpu.VMEM_SHARED((N_PAD, IN_C), jnp.float32),
    ],
    compiler_params=_no_tc_tiling,
)
def _agg_kernel(ei2_hbm, xs_hbm, z2_hbm, out_hbm, siA, diA, rA, siB, diB,
                rB, stage, semA, semB, semG, semS, acc):
    c = lax.axis_index("c")
    s = lax.axis_index("s")
    w = s * NC + c
    row0 = w * ROWS_PT
    nblk = jnp.minimum(ROWS_PT, DROW - w * ROWS_PT) // SB2
    pltpu.sync_copy(z2_hbm, stage)
    pltpu.sync_copy(stage, acc.at[pl.ds(s * STRIPE, STRIPE), :])
    plsc.subcore_barrier()

    def fetch(blk, si, di, sem):
        pltpu.async_copy(ei2_hbm.at[pl.ds(row0 + blk * SB2, SB2), :], si, sem)
        pltpu.async_copy(ei2_hbm.at[pl.ds(DROW + row0 + blk * SB2, SB2), :],
                         di, sem)

    def process(blk, si, di, rr, sem):
        pltpu.make_async_copy(ei2_hbm.at[pl.ds(0, SB2), :], si, sem).wait()
        pltpu.make_async_copy(ei2_hbm.at[pl.ds(0, SB2), :], di, sem).wait()
        gat = [pltpu.async_copy(xs_hbm.at[si.at[k]], rr.at[k], semG)
               for k in range(SB2)]
        scat = []
        for k in range(SB2):
            gat[k].wait()
            scat.append(pltpu.async_copy(rr.at[k], acc.at[di.at[k]], semS,
                                         add=True))
        for d in scat:
            d.wait()

        @pl.when(blk + 2 < nblk)
        def _():
            fetch(blk + 2, si, di, sem)

    fetch(0, siA, diA, semA)
    fetch(1, siB, diB, semB)

    def body(i, carry):
        process(2 * i, siA, diA, rA, semA)
        process(2 * i + 1, siB, diB, rB, semB)
        return carry

    lax.fori_loop(0, nblk // 2, body, 0)

    @pl.when(nblk % 2 == 1)
    def _():
        process(nblk - 1, siA, diA, rA, semA)

    plsc.subcore_barrier()
    pltpu.sync_copy(acc.at[pl.ds(s * STRIPE, STRIPE), :], stage)
    pltpu.sync_copy(stage, out_hbm.at[c, pl.ds(s * STRIPE, STRIPE), :])


def _prep_body(cnt2_ref, x_ref, dinv_ref, xs_ref):
    cnt = cnt2_ref[0] + cnt2_ref[1]
    dinv = lax.rsqrt(cnt + 1.0)
    dinv_ref[...] = dinv
    xs_ref[...] = x_ref[...] * dinv


def _tail_body(t2_ref, xs_ref, dinv_ref, mz_ref, cz_ref, mh_ref, ch_ref,
               w_ref, b_ref, out_ref):
    s = dinv_ref[...] * (t2_ref[0] + t2_ref[1] + xs_ref[...])
    gz = jnp.dot(s, mz_ref[...], preferred_element_type=jnp.float32,
                 precision=lax.Precision.HIGHEST) + cz_ref[...]
    gh = jnp.dot(s, mh_ref[...], preferred_element_type=jnp.float32,
                 precision=lax.Precision.HIGHEST) + ch_ref[...]
    z = jax.nn.sigmoid(gz)
    ht = jnp.tanh(gh)
    y = (1.0 - z) * ht
    out_ref[...] = jnp.dot(y, w_ref[...], preferred_element_type=jnp.float32,
                           precision=lax.Precision.HIGHEST) + b_ref[...]


def kernel(x, edge_index, Wz, bz, Wr, br, Wh, bh, lzW, lzb, lrW, lrb, lhW,
           lhb, linW, linb):
    n = x.shape[0]
    assert n == N_NODES and edge_index.shape[1] == E

    Az = lzW[:, :HID].T
    Ah = lhW[:, :HID].T
    Mz = Wz @ Az
    cz = (bz @ Az + lzb).reshape(1, HID)
    Mh = Wh @ Ah
    ch = (bh @ Ah + lhb).reshape(1, HID)
    wv = linW.reshape(HID, 1)
    bs = linb.reshape(1, 1)

    x_pad = jnp.zeros((N_PAD, IN_C), jnp.float32).at[:n].set(x)
    z1 = jnp.zeros((STRIPE,), jnp.float32)
    z2 = jnp.zeros((STRIPE, IN_C), jnp.float32)
    ones = jnp.ones((CHUNK,), jnp.float32)

    ei2 = edge_index.reshape(2 * DROW, CHUNK)

    cnt2 = _count_kernel(ei2, z1, ones)

    grid = (N_PAD // NB,)
    dinv, xs = pl.pallas_call(
        _prep_body,
        grid=grid,
        in_specs=[
            pl.BlockSpec((NC, NB, 1), lambda i: (0, i, 0)),
            pl.BlockSpec((NB, IN_C), lambda i: (i, 0)),
        ],
        out_specs=[
            pl.BlockSpec((NB, 1), lambda i: (i, 0)),
            pl.BlockSpec((NB, IN_C), lambda i: (i, 0)),
        ],
        out_shape=[
            jax.ShapeDtypeStruct((N_PAD, 1), jnp.float32),
            jax.ShapeDtypeStruct((N_PAD, IN_C), jnp.float32),
        ],
    )(cnt2.reshape(NC, N_PAD, 1), x_pad)

    t2 = _agg_kernel(ei2, xs, z2)

    out = pl.pallas_call(
        _tail_body,
        grid=grid,
        in_specs=[
            pl.BlockSpec((NC, NB, IN_C), lambda i: (0, i, 0)),
            pl.BlockSpec((NB, IN_C), lambda i: (i, 0)),
            pl.BlockSpec((NB, 1), lambda i: (i, 0)),
            pl.BlockSpec((IN_C, HID), lambda i: (0, 0)),
            pl.BlockSpec((1, HID), lambda i: (0, 0)),
            pl.BlockSpec((IN_C, HID), lambda i: (0, 0)),
            pl.BlockSpec((1, HID), lambda i: (0, 0)),
            pl.BlockSpec((HID, 1), lambda i: (0, 0)),
            pl.BlockSpec((1, 1), lambda i: (0, 0)),
        ],
        out_specs=pl.BlockSpec((NB, 1), lambda i: (i, 0)),
        out_shape=jax.ShapeDtypeStruct((N_PAD, 1), jnp.float32),
    )(t2, xs, dinv, Mz, cz, Mh, ch, wv, bs)

    return out[:n, 0]


# speedup vs baseline: 203.9043x; 1.7807x over previous
import functools

import jax
import jax.numpy as jnp
from jax import lax
from jax.experimental import pallas as pl
from jax.experimental.pallas import tpu as pltpu
from jax.experimental.pallas import tpu_sc as plsc

N_NODES = 100000
IN_C = 8
HID = 32

NC = 2
NS = 16
NW = NC * NS

CHUNK = 512
N_PAD = 100096
STRIPE = N_PAD // NS
NP16 = N_PAD // 16
NPL = N_PAD // 128
TCG = 17
RB = NP16 // TCG
CB = NPL // TCG

E = 3200000
DROW = E // CHUNK
ROWS_PT = 196
SB2 = 2
SB1 = 2

_mesh = plsc.VectorSubcoreMesh(core_axis_name="c", subcore_axis_name="s")
_no_tc_tiling = pltpu.CompilerParams(use_tc_tiling_on_sc=False)


@functools.partial(
    pl.kernel,
    out_type=jax.ShapeDtypeStruct((NC * N_PAD,), jnp.float32),
    mesh=_mesh,
    scratch_types=[
        pltpu.VMEM((SB1, CHUNK), jnp.int32),
        pltpu.VMEM((SB1, CHUNK), jnp.int32),
        pltpu.VMEM((CHUNK,), jnp.float32),
        pltpu.VMEM((STRIPE,), jnp.float32),
        pltpu.SemaphoreType.DMA,
        pltpu.SemaphoreType.DMA,
        pltpu.SemaphoreType.DMA,
        pltpu.VMEM_SHARED((N_PAD,), jnp.float32),
    ],
    compiler_params=_no_tc_tiling,
)
def _count_kernel(ei2_hbm, z1_hbm, ones_hbm, cnt_hbm, diA, diB, ones,
                  stage, semA, semB, semS, cnt_sh):
    c = lax.axis_index("c")
    s = lax.axis_index("s")
    w = s * NC + c
    row0 = DROW + w * ROWS_PT
    nblk = jnp.minimum(ROWS_PT, DROW - w * ROWS_PT) // SB1
    pltpu.sync_copy(z1_hbm, stage)
    pltpu.sync_copy(stage, cnt_sh.at[pl.ds(s * STRIPE, STRIPE)])
    pltpu.sync_copy(ones_hbm, ones)
    plsc.subcore_barrier()

    def fetch(blk, di, sem):
        pltpu.async_copy(ei2_hbm.at[pl.ds(row0 + blk * SB1, SB1), :], di, sem)

    def process(blk, di, sem):
        pltpu.make_async_copy(ei2_hbm.at[pl.ds(0, SB1), :], di, sem).wait()
        scat = [pltpu.async_copy(ones, cnt_sh.at[di.at[k]], semS, add=True)
                for k in range(SB1)]
        for d in scat:
            d.wait()

        @pl.when(blk + 2 < nblk)
        def _():
            fetch(blk + 2, di, sem)

    fetch(0, diA, semA)
    fetch(1, diB, semB)

    def body(i, carry):
        process(2 * i, diA, semA)
        process(2 * i + 1, diB, semB)
        return carry

    lax.fori_loop(0, nblk // 2, body, 0)

    @pl.when(nblk % 2 == 1)
    def _():
        process(nblk - 1, diA, semA)

    plsc.subcore_barrier()
    pltpu.sync_copy(cnt_sh.at[pl.ds(s * STRIPE, STRIPE)], stage)
    pltpu.sync_copy(stage, cnt_hbm.at[pl.ds(c * N_PAD + s * STRIPE, STRIPE)])


@functools.partial(
    pl.kernel,
    out_type=jax.ShapeDtypeStruct((NC, N_PAD, IN_C), jnp.float32),
    mesh=_mesh,
    scratch_types=[
        pltpu.VMEM((SB2, CHUNK), jnp.int32),
        pltpu.VMEM((SB2, CHUNK), jnp.int32),
        pltpu.VMEM((SB2, CHUNK, IN_C), jnp.float32),
        pltpu.VMEM((SB2, CHUNK), jnp.int32),
        pltpu.VMEM((SB2, CHUNK), jnp.int32),
        pltpu.VMEM((SB2, CHUNK, IN_C), jnp.float32),
        pltpu.VMEM((STRIPE, IN_C), jnp.float32),
        pltpu.SemaphoreType.DMA,
        pltpu.SemaphoreType.DMA,
        pltpu.SemaphoreType.DMA,
        pltpu.SemaphoreType.DMA,
        pltpu.VMEM_SHARED((N_PAD, IN_C), jnp.float32),
    ],
    compiler_params=_no_tc_tiling,
)
def _agg_kernel(ei2_hbm, xs_hbm, z2_hbm, out_hbm, siA, diA, rA, siB, diB,
                rB, stage, semA, semB, semG, semS, acc):
    c = lax.axis_index("c")
    s = lax.axis_index("s")
    w = s * NC + c
    row0 = w * ROWS_PT
    nblk = jnp.minimum(ROWS_PT, DROW - w * ROWS_PT) // SB2
    pltpu.sync_copy(z2_hbm, stage)
    pltpu.sync_copy(stage, acc.at[pl.ds(s * STRIPE, STRIPE), :])
    plsc.subcore_barrier()

    def fetch(blk, si, di, sem):
        pltpu.async_copy(ei2_hbm.at[pl.ds(row0 + blk * SB2, SB2), :], si, sem)
        pltpu.async_copy(ei2_hbm.at[pl.ds(DROW + row0 + blk * SB2, SB2), :],
                         di, sem)

    def process(blk, si, di, rr, sem):
        pltpu.make_async_copy(ei2_hbm.at[pl.ds(0, SB2), :], si, sem).wait()
        pltpu.make_async_copy(ei2_hbm.at[pl.ds(0, SB2), :], di, sem).wait()
        gat = [pltpu.async_copy(xs_hbm.at[si.at[k]], rr.at[k], semG)
               for k in range(SB2)]
        scat = []
        for k in range(SB2):
            gat[k].wait()
            scat.append(pltpu.async_copy(rr.at[k], acc.at[di.at[k]], semS,
                                         add=True))
        for d in scat:
            d.wait()

        @pl.when(blk + 2 < nblk)
        def _():
            fetch(blk + 2, si, di, sem)

    fetch(0, siA, diA, semA)
    fetch(1, siB, diB, semB)

    def body(i, carry):
        process(2 * i, siA, diA, rA, semA)
        process(2 * i + 1, siB, diB, rB, semB)
        return carry

    lax.fori_loop(0, nblk // 2, body, 0)

    @pl.when(nblk % 2 == 1)
    def _():
        process(nblk - 1, siA, diA, rA, semA)

    plsc.subcore_barrier()
    pltpu.sync_copy(acc.at[pl.ds(s * STRIPE, STRIPE), :], stage)
    pltpu.sync_copy(stage, out_hbm.at[c, pl.ds(s * STRIPE, STRIPE), :])


def _tail_body(tpk_ref, xs_ref, dinv_ref, w1_ref, c1_ref, w2_ref, b_ref,
               out_ref):
    s = dinv_ref[...] * (tpk_ref[0] + tpk_ref[1] + xs_ref[...])
    g = jnp.dot(s, w1_ref[...], preferred_element_type=jnp.float32,
                precision=lax.Precision.HIGHEST) + c1_ref[...]
    y = (1.0 - jax.nn.sigmoid(g[:, :512])) * jnp.tanh(g[:, 512:])
    out_ref[...] = jnp.dot(y, w2_ref[...], preferred_element_type=jnp.float32,
                           precision=lax.Precision.HIGHEST) + b_ref[...]


def kernel(x, edge_index, Wz, bz, Wr, br, Wh, bh, lzW, lzb, lrW, lrb, lhW,
           lhb, linW, linb):
    n = x.shape[0]
    assert n == N_NODES and edge_index.shape[1] == E

    Az = lzW[:, :HID].T
    Ah = lhW[:, :HID].T
    Mz = Wz @ Az
    Mh = Wh @ Ah
    cz = bz @ Az + lzb
    ch = bh @ Ah + lhb
    eye16 = jnp.eye(16, dtype=jnp.float32)
    W1 = jnp.concatenate([jnp.kron(eye16, Mz), jnp.kron(eye16, Mh)], axis=1)
    c1 = jnp.concatenate([jnp.tile(cz, 16), jnp.tile(ch, 16)]).reshape(1, 1024)
    W2 = jnp.kron(eye16, linW.reshape(HID, 1))
    bs = jnp.full((1, 16), linb[0], jnp.float32)

    x_flat = jnp.pad(x.reshape(-1), (0, (N_PAD - n) * IN_C))
    z1 = jnp.zeros((STRIPE,), jnp.float32)
    z2 = jnp.zeros((STRIPE, IN_C), jnp.float32)
    ones = jnp.ones((CHUNK,), jnp.float32)

    ei2 = edge_index.reshape(2 * DROW, CHUNK)

    cnt2 = _count_kernel(ei2, z1, ones)

    dinv = lax.rsqrt(cnt2[:N_PAD] + cnt2[N_PAD:] + 1.0)
    dexp = jnp.repeat(dinv, IN_C)
    xs_flat = x_flat * dexp

    t2 = _agg_kernel(ei2, xs_flat.reshape(N_PAD, IN_C), z2)

    out = pl.pallas_call(
        _tail_body,
        grid=(TCG,),
        in_specs=[
            pl.BlockSpec((NC, RB, 128), lambda i: (0, i, 0)),
            pl.BlockSpec((RB, 128), lambda i: (i, 0)),
            pl.BlockSpec((RB, 128), lambda i: (i, 0)),
            pl.BlockSpec((128, 1024), lambda i: (0, 0)),
            pl.BlockSpec((1, 1024), lambda i: (0, 0)),
            pl.BlockSpec((512, 16), lambda i: (0, 0)),
            pl.BlockSpec((1, 16), lambda i: (0, 0)),
        ],
        out_specs=pl.BlockSpec((RB, 16), lambda i: (i, 0)),
        out_shape=jax.ShapeDtypeStruct((NP16, 16), jnp.float32),
    )(t2.reshape(NC, NP16, 128), xs_flat.reshape(NP16, 128),
      dexp.reshape(NP16, 128), W1, c1, W2, bs)

    return out.reshape(N_PAD)[:n]
